# baseline (speedup 1.0000x reference)
import functools

import jax
import jax.numpy as jnp
from jax import lax
from jax.experimental import pallas as pl
from jax.experimental.pallas import tpu as pltpu
from jax.experimental.pallas import tpu_sc as plsc

VOCAB = 1000000
EMB_DIM = 64
B = 4096
L = 200
EPS = 1e-12

NW = 32
CHUNK = 128
ROWS_PER_W = (B * L) // NW
NCHUNK = ROWS_PER_W // CHUNK
GROUPS = CHUNK // 16


def _ln_chunk(in_ref, out_ref, gamma_ref, beta_ref):
    def group_body(g, carry):
        row_ids = g * 16 + lax.iota(jnp.int32, 16)
        s = jnp.zeros((16,), jnp.float32)
        ss = jnp.zeros((16,), jnp.float32)
        for j in range(EMB_DIM):
            colj = jnp.full((16,), j, jnp.int32)
            v = plsc.load_gather(in_ref, [row_ids, colj])
            s = s + v
            ss = ss + v * v
        mean = s * (1.0 / EMB_DIM)
        var = ss * (1.0 / EMB_DIM) - mean * mean
        x = var + EPS
        xi = lax.bitcast_convert_type(x, jnp.int32)
        y = lax.bitcast_convert_type(jnp.int32(0x5F3759DF) - (xi >> 1),
                                     jnp.float32)
        for _ in range(3):
            y = y * (1.5 - 0.5 * x * y * y)
        rstd = y
        for j in range(EMB_DIM):
            colj = jnp.full((16,), j, jnp.int32)
            v = plsc.load_gather(in_ref, [row_ids, colj])
            g_j = plsc.load_gather(gamma_ref, [colj])
            b_j = plsc.load_gather(beta_ref, [colj])
            o = (v - mean) * rstd * g_j + b_j
            plsc.store_scatter(out_ref, [row_ids, colj], o)
        return carry

    lax.fori_loop(0, GROUPS, group_body, 0)


def _make_kernel():
    mesh = plsc.VectorSubcoreMesh(core_axis_name="c", subcore_axis_name="s")

    @functools.partial(
        pl.kernel,
        mesh=mesh,
        out_type=jax.ShapeDtypeStruct((B * L, EMB_DIM), jnp.float32),
        compiler_params=pltpu.CompilerParams(
            use_tc_tiling_on_sc=False,
            needs_layout_passes=False,
        ),
        scratch_types=[
            pltpu.VMEM((NCHUNK, CHUNK), jnp.int32),
            pltpu.VMEM((CHUNK, EMB_DIM), jnp.float32),
            pltpu.VMEM((CHUNK, EMB_DIM), jnp.float32),
            pltpu.VMEM((CHUNK, EMB_DIM), jnp.float32),
            pltpu.VMEM((CHUNK, EMB_DIM), jnp.float32),
            pltpu.VMEM((EMB_DIM,), jnp.float32),
            pltpu.VMEM((EMB_DIM,), jnp.float32),
            pltpu.SemaphoreType.DMA,
            pltpu.SemaphoreType.DMA,
            pltpu.SemaphoreType.DMA,
            pltpu.SemaphoreType.DMA,
        ],
    )
    def kern(ids_hbm, table_hbm, gamma_hbm, beta_hbm, out_hbm,
             idx_v, in0, in1, out0, out1, gamma_v, beta_v,
             gsem0, gsem1, osem0, osem1):
        wid = lax.axis_index("s") * 2 + lax.axis_index("c")
        wbase = wid * ROWS_PER_W

        pltpu.sync_copy(gamma_hbm, gamma_v)
        pltpu.sync_copy(beta_hbm, beta_v)
        pltpu.sync_copy(ids_hbm.at[wid], idx_v)

        ins = (in0, in1)
        outs = (out0, out1)
        gsems = (gsem0, gsem1)
        osems = (osem0, osem1)

        def gather_start(k, b):
            pltpu.async_copy(table_hbm.at[idx_v.at[k]], ins[b], gsems[b])

        def gather_wait(k, b):
            pltpu.make_async_copy(table_hbm.at[idx_v.at[k]], ins[b],
                                  gsems[b]).wait()

        def out_start(k, b):
            pltpu.async_copy(outs[b],
                             out_hbm.at[pl.ds(wbase + k * CHUNK, CHUNK)],
                             osems[b])

        def out_wait(k, b):
            pltpu.make_async_copy(outs[b],
                                  out_hbm.at[pl.ds(wbase + k * CHUNK, CHUNK)],
                                  osems[b]).wait()

        gather_start(0, 0)
        gather_start(1, 1)

        def body(i, carry):
            for b in range(2):
                k = 2 * i + b
                gather_wait(k, b)

                @pl.when(k >= 2)
                def _():
                    out_wait(k - 2, b)

                _ln_chunk(ins[b], outs[b], gamma_v, beta_v)
                out_start(k, b)

                @pl.when(k + 2 < NCHUNK)
                def _():
                    gather_start(k + 2, b)
            return carry

        lax.fori_loop(0, NCHUNK // 2, body, 0)

        out_wait(NCHUNK - 2, 0)
        out_wait(NCHUNK - 1, 1)

    return kern


_KERNEL = _make_kernel()


@jax.jit
def kernel(input_ids, table, ln_gamma, ln_beta):
    ids = input_ids.reshape(NW, NCHUNK, CHUNK)
    out = _KERNEL(ids, table, ln_gamma, ln_beta)
    return out.reshape(B, L, EMB_DIM)
